# Initial kernel scaffold; baseline (speedup 1.0000x reference)
import jax
import jax.numpy as jnp
from jax.experimental import pallas as pl


def kernel(prog_low_w, prog_low_bn_s, prog_low_bn_b, low_dw_w, low_dw_bn_s, low_dw_bn_b, prog_high_w, prog_high_bn_s, prog_high_bn_b, high_dw_w, high_dw_bn_s, high_dw_bn_b, gather_w, gather_bn_s, gather_bn_b, low, high):
    raise NotImplementedError("write your pallas kernel here")


# speedup vs baseline: 1.0415x; 1.0415x over previous
import functools

import numpy as np
import jax
import jax.numpy as jnp
from jax.experimental import pallas as pl
from jax.experimental.pallas import tpu as pltpu


def _bilinear_matrix(out_size, in_size):
    if out_size == 1 or in_size == 1:
        R = np.zeros((out_size, in_size), np.float32)
        R[:, 0] = 1.0
        return R
    src = np.arange(out_size, dtype=np.float64) * (in_size - 1) / (out_size - 1)
    lo = np.clip(np.floor(src).astype(np.int64), 0, in_size - 1)
    hi = np.clip(lo + 1, 0, in_size - 1)
    frac = (src - lo).astype(np.float32)
    R = np.zeros((out_size, in_size), np.float32)
    R[np.arange(out_size), lo] += 1.0 - frac
    R[np.arange(out_size), hi] += frac
    return R


def _branch_kernel(x_ref, wpw_ref, bpw_ref, wdw_ref, bdw_ref, wproj_ref,
                   o_ref, pad_ref, *, H, W):
    C = wpw_ref.shape[1]

    y = jnp.dot(x_ref[0].astype(jnp.bfloat16), wpw_ref[...],
                preferred_element_type=jnp.float32)
    y = jnp.maximum(y + bpw_ref[...], 0.0)

    pad_ref[...] = jnp.zeros_like(pad_ref)
    pad_ref[pl.ds(1, H), pl.ds(8, W), :] = y.reshape(H, W, C)
    xp = pad_ref[...]

    acc = jnp.zeros((H, W, C), jnp.float32)
    for dy in range(3):
        for dx in range(3):
            acc = acc + xp[dy:dy + H, dx + 7:dx + 7 + W, :] * wdw_ref[dy * 3 + dx, :]
    z = jnp.maximum(acc + bdw_ref[0], 0.0)

    o_ref[0] = jnp.dot(z.reshape(H * W, C).astype(jnp.bfloat16), wproj_ref[...],
                       preferred_element_type=jnp.float32).astype(o_ref.dtype)


def _run_branch(x2d, H, W, wpw, bpw, wdw, bdw, wproj, out_dtype):
    N, HW, Cin = x2d.shape
    Cmid = wpw.shape[1]
    Cout = wproj.shape[1]
    flops = int(N * HW * (2 * Cin * Cmid + 18 * Cmid + 2 * Cmid * Cout))
    bytes_accessed = int(4 * x2d.size + N * HW * Cout * out_dtype.itemsize
                         + 2 * (wpw.size + wproj.size)
                         + 4 * (bpw.size + wdw.size + bdw.size))
    return pl.pallas_call(
        functools.partial(_branch_kernel, H=H, W=W),
        out_shape=jax.ShapeDtypeStruct((N, HW, Cout), out_dtype),
        grid=(N,),
        in_specs=[
            pl.BlockSpec((1, HW, Cin), lambda b: (b, 0, 0)),
            pl.BlockSpec((Cin, Cmid), lambda b: (0, 0)),
            pl.BlockSpec((1, Cmid), lambda b: (0, 0)),
            pl.BlockSpec((9, Cmid), lambda b: (0, 0)),
            pl.BlockSpec((1, Cmid), lambda b: (0, 0)),
            pl.BlockSpec((Cmid, Cout), lambda b: (0, 0)),
        ],
        out_specs=pl.BlockSpec((1, HW, Cout), lambda b: (b, 0, 0)),
        scratch_shapes=[pltpu.VMEM((H + 2, W + 16, Cmid), jnp.float32)],
        compiler_params=pltpu.CompilerParams(
            dimension_semantics=("parallel",),
            vmem_limit_bytes=64 * 1024 * 1024),
        cost_estimate=pl.CostEstimate(flops=flops, transcendentals=0,
                                      bytes_accessed=bytes_accessed),
    )(x2d, wpw, bpw, wdw, bdw, wproj)


def _combine_kernel(gl_ref, gh_ref, rh_ref, rwe_ref, bias_ref, o_ref):
    u = jnp.dot(rh_ref[...], gh_ref[0], preferred_element_type=jnp.float32)
    v = jnp.dot(u.astype(jnp.bfloat16), rwe_ref[...],
                preferred_element_type=jnp.float32)
    o_ref[0] = (gl_ref[0] + v + bias_ref[...]).astype(o_ref.dtype)


def _run_combine(gl2, gh2, Rh, RwExp, bias_row):
    N, Hl, WlC = gl2.shape
    _, Hh, WhC = gh2.shape
    flops = int(N * (2 * Hl * Hh * WhC + 2 * Hl * WhC * WlC))
    bytes_accessed = int(4 * gl2.size + 2 * gh2.size + 2 * Rh.size
                         + 2 * RwExp.size + 4 * bias_row.size
                         + 4 * N * Hl * WlC)
    return pl.pallas_call(
        _combine_kernel,
        out_shape=jax.ShapeDtypeStruct((N, Hl, WlC), jnp.float32),
        grid=(N,),
        in_specs=[
            pl.BlockSpec((1, Hl, WlC), lambda b: (b, 0, 0)),
            pl.BlockSpec((1, Hh, WhC), lambda b: (b, 0, 0)),
            pl.BlockSpec((Hl, Hh), lambda b: (0, 0)),
            pl.BlockSpec((WhC, WlC), lambda b: (0, 0)),
            pl.BlockSpec((1, WlC), lambda b: (0, 0)),
        ],
        out_specs=pl.BlockSpec((1, Hl, WlC), lambda b: (b, 0, 0)),
        compiler_params=pltpu.CompilerParams(
            dimension_semantics=("parallel",),
            vmem_limit_bytes=64 * 1024 * 1024),
        cost_estimate=pl.CostEstimate(flops=flops, transcendentals=0,
                                      bytes_accessed=bytes_accessed),
    )(gl2, gh2, Rh, RwExp, bias_row)


def kernel(prog_low_w, prog_low_bn_s, prog_low_bn_b,
           low_dw_w, low_dw_bn_s, low_dw_bn_b,
           prog_high_w, prog_high_bn_s, prog_high_bn_b,
           high_dw_w, high_dw_bn_s, high_dw_bn_b,
           gather_w, gather_bn_s, gather_bn_b,
           low, high):
    N, Cl, Hl, Wl = low.shape
    _, Ch, Hh, Wh = high.shape
    mid_l = prog_low_w.shape[1]
    dim_out = gather_w.shape[1]

    wpw_low = (prog_low_w * prog_low_bn_s[None, :]).astype(jnp.bfloat16)
    wpw_high = (prog_high_w * prog_high_bn_s[None, :]).astype(jnp.bfloat16)
    wdw_low = low_dw_w * low_dw_bn_s[None, :]
    wdw_high = high_dw_w * high_dw_bn_s[None, :]
    wg = gather_w * gather_bn_s[None, :]
    wproj_low = wg[:mid_l].astype(jnp.bfloat16)
    wproj_high = wg[mid_l:].astype(jnp.bfloat16)

    low2d = jnp.transpose(low, (0, 2, 3, 1)).reshape(N, Hl * Wl, Cl)
    high2d = jnp.transpose(high, (0, 2, 3, 1)).reshape(N, Hh * Wh, Ch)

    gl = _run_branch(low2d, Hl, Wl, wpw_low, prog_low_bn_b.reshape(1, -1),
                     wdw_low, low_dw_bn_b.reshape(1, -1), wproj_low,
                     jnp.dtype(jnp.float32))
    gh = _run_branch(high2d, Hh, Wh, wpw_high, prog_high_bn_b.reshape(1, -1),
                     wdw_high, high_dw_bn_b.reshape(1, -1), wproj_high,
                     jnp.dtype(jnp.bfloat16))

    Rh = jnp.asarray(_bilinear_matrix(Hl, Hh).astype(np.float32)).astype(jnp.bfloat16)
    Rw_np = _bilinear_matrix(Wl, Wh)
    RwExp = jnp.asarray(
        np.kron(Rw_np.T, np.eye(dim_out, dtype=np.float32))).astype(jnp.bfloat16)
    bias_row = jnp.tile(gather_bn_b, Wl).reshape(1, Wl * dim_out)

    gl2 = gl.reshape(N, Hl, Wl * dim_out)
    gh2 = gh.reshape(N, Hh, Wh * dim_out)
    out2 = _run_combine(gl2, gh2, Rh, RwExp, bias_row)

    out = out2.reshape(N, Hl, Wl, dim_out)
    return jnp.transpose(out, (0, 3, 1, 2))


# speedup vs baseline: 1.4378x; 1.3804x over previous
import functools

import numpy as np
import jax
import jax.numpy as jnp
from jax.experimental import pallas as pl
from jax.experimental.pallas import tpu as pltpu


def _bilinear_matrix(out_size, in_size):
    if out_size == 1 or in_size == 1:
        R = np.zeros((out_size, in_size), np.float32)
        R[:, 0] = 1.0
        return R
    src = np.arange(out_size, dtype=np.float64) * (in_size - 1) / (out_size - 1)
    lo = np.clip(np.floor(src).astype(np.int64), 0, in_size - 1)
    hi = np.clip(lo + 1, 0, in_size - 1)
    frac = (src - lo).astype(np.float32)
    R = np.zeros((out_size, in_size), np.float32)
    R[np.arange(out_size), lo] += 1.0 - frac
    R[np.arange(out_size), hi] += frac
    return R


def _branch(x_ref, wpw_ref, bpw_ref, wdw_ref, bdw_ref, wproj_ref, bout_ref,
            o_ref, pad_ref, sl_ref, sr_ref, *, H, W):
    C = wpw_ref.shape[1]
    y = jax.lax.dot_general(
        x_ref[0].astype(jnp.bfloat16), wpw_ref[...],
        dimension_numbers=(((0,), (0,)), ((), ())),
        preferred_element_type=jnp.float32)
    y = jnp.maximum(y + bpw_ref[...], 0.0)

    pad_ref[...] = jnp.zeros_like(pad_ref)
    pad_ref[pl.ds(1, H), pl.ds(8, W), :] = y.reshape(H, W, C)
    xp = pad_ref[...]

    sl_ref[...] = xp[:, 7:7 + W, :]
    sr_ref[...] = xp[:, 9:9 + W, :]
    left = sl_ref[...]
    right = sr_ref[...]
    acc = jnp.zeros((H, W, C), jnp.float32)
    for dy in range(3):
        acc = acc + (left[dy:dy + H] * wdw_ref[dy * 3 + 0]
                     + xp[dy:dy + H, 8:8 + W, :] * wdw_ref[dy * 3 + 1]
                     + right[dy:dy + H] * wdw_ref[dy * 3 + 2])
    z = jnp.maximum(acc + bdw_ref[0], 0.0)

    g = jax.lax.dot_general(
        wproj_ref[...], z.reshape(H * W, C).astype(jnp.bfloat16),
        dimension_numbers=(((0,), (1,)), ((), ())),
        preferred_element_type=jnp.float32)
    o_ref[0] = (g + bout_ref[...]).astype(o_ref.dtype)


def _branches_kernel(low_ref, high_ref,
                     wpwl_ref, bpwl_ref, wdwl_ref, bdwl_ref, wprl_ref, bg_ref,
                     wpwh_ref, bpwh_ref, wdwh_ref, bdwh_ref, wprh_ref, z0_ref,
                     gl_ref, gh_ref, padl_ref, padh_ref, sll_ref, srl_ref,
                     slh_ref, srh_ref, *, Hl, Wl, Hh, Wh):
    _branch(low_ref, wpwl_ref, bpwl_ref, wdwl_ref, bdwl_ref, wprl_ref, bg_ref,
            gl_ref, padl_ref, sll_ref, srl_ref, H=Hl, W=Wl)
    _branch(high_ref, wpwh_ref, bpwh_ref, wdwh_ref, bdwh_ref, wprh_ref, z0_ref,
            gh_ref, padh_ref, slh_ref, srh_ref, H=Hh, W=Wh)


def _run_branches(low3, high3, wl, bl, dwl, dbl, pl_w, bias_g,
                  wh, bh, dwh, dbh, ph_w, Hl, Wl, Hh, Wh, Cout):
    N, Cl, HWl = low3.shape
    _, Ch, HWh = high3.shape
    Cml = wl.shape[1]
    Cmh = wh.shape[1]
    flops = int(N * (HWl * (2 * Cl * Cml + 18 * Cml + 2 * Cml * Cout)
                     + HWh * (2 * Ch * Cmh + 18 * Cmh + 2 * Cmh * Cout)))
    bytes_accessed = int(4 * (low3.size + high3.size)
                         + 4 * N * Cout * HWl + 2 * N * Cout * HWh)
    zero_bias = jnp.zeros((Cout, 1), jnp.float32)
    return pl.pallas_call(
        functools.partial(_branches_kernel, Hl=Hl, Wl=Wl, Hh=Hh, Wh=Wh),
        out_shape=(jax.ShapeDtypeStruct((N, Cout, HWl), jnp.float32),
                   jax.ShapeDtypeStruct((N, Cout, HWh), jnp.bfloat16)),
        grid=(N,),
        in_specs=[
            pl.BlockSpec((1, Cl, HWl), lambda b: (b, 0, 0)),
            pl.BlockSpec((1, Ch, HWh), lambda b: (b, 0, 0)),
            pl.BlockSpec((Cl, Cml), lambda b: (0, 0)),
            pl.BlockSpec((1, Cml), lambda b: (0, 0)),
            pl.BlockSpec((9, Cml), lambda b: (0, 0)),
            pl.BlockSpec((1, Cml), lambda b: (0, 0)),
            pl.BlockSpec((Cml, Cout), lambda b: (0, 0)),
            pl.BlockSpec((Cout, 1), lambda b: (0, 0)),
            pl.BlockSpec((Ch, Cmh), lambda b: (0, 0)),
            pl.BlockSpec((1, Cmh), lambda b: (0, 0)),
            pl.BlockSpec((9, Cmh), lambda b: (0, 0)),
            pl.BlockSpec((1, Cmh), lambda b: (0, 0)),
            pl.BlockSpec((Cmh, Cout), lambda b: (0, 0)),
            pl.BlockSpec((Cout, 1), lambda b: (0, 0)),
        ],
        out_specs=(pl.BlockSpec((1, Cout, HWl), lambda b: (b, 0, 0)),
                   pl.BlockSpec((1, Cout, HWh), lambda b: (b, 0, 0))),
        scratch_shapes=[pltpu.VMEM((Hl + 2, Wl + 16, Cml), jnp.float32),
                        pltpu.VMEM((Hh + 2, Wh + 16, Cmh), jnp.float32),
                        pltpu.VMEM((Hl + 2, Wl, Cml), jnp.float32),
                        pltpu.VMEM((Hl + 2, Wl, Cml), jnp.float32),
                        pltpu.VMEM((Hh + 2, Wh, Cmh), jnp.float32),
                        pltpu.VMEM((Hh + 2, Wh, Cmh), jnp.float32)],
        compiler_params=pltpu.CompilerParams(
            dimension_semantics=("parallel",),
            vmem_limit_bytes=64 * 1024 * 1024),
        cost_estimate=pl.CostEstimate(flops=flops, transcendentals=0,
                                      bytes_accessed=bytes_accessed),
    )(low3, high3, wl, bl, dwl, dbl, pl_w, bias_g,
      wh, bh, dwh, dbh, ph_w, zero_bias)


def _combine_kernel(gl_ref, gh_ref, mt_ref, o_ref, *, B, Cout):
    HWh = gh_ref.shape[2]
    HWl = gl_ref.shape[2]
    gh = gh_ref[...].reshape(B * Cout, HWh)
    r = jnp.dot(gh, mt_ref[...], preferred_element_type=jnp.float32)
    o_ref[...] = (gl_ref[...].reshape(B * Cout, HWl) + r).reshape(
        B, Cout, HWl).astype(o_ref.dtype)


def _run_combine(gl, gh, MT, batch):
    N, Cout, HWl = gl.shape
    _, _, HWh = gh.shape
    flops = int(2 * N * Cout * HWh * HWl)
    bytes_accessed = int(4 * gl.size + 2 * gh.size + 2 * MT.size + 4 * gl.size)
    return pl.pallas_call(
        functools.partial(_combine_kernel, B=batch, Cout=Cout),
        out_shape=jax.ShapeDtypeStruct((N, Cout, HWl), jnp.float32),
        grid=(N // batch,),
        in_specs=[
            pl.BlockSpec((batch, Cout, HWl), lambda b: (b, 0, 0)),
            pl.BlockSpec((batch, Cout, HWh), lambda b: (b, 0, 0)),
            pl.BlockSpec((HWh, HWl), lambda b: (0, 0)),
        ],
        out_specs=pl.BlockSpec((batch, Cout, HWl), lambda b: (b, 0, 0)),
        compiler_params=pltpu.CompilerParams(
            dimension_semantics=("parallel",),
            vmem_limit_bytes=64 * 1024 * 1024),
        cost_estimate=pl.CostEstimate(flops=flops, transcendentals=0,
                                      bytes_accessed=bytes_accessed),
    )(gl, gh, MT)


def kernel(prog_low_w, prog_low_bn_s, prog_low_bn_b,
           low_dw_w, low_dw_bn_s, low_dw_bn_b,
           prog_high_w, prog_high_bn_s, prog_high_bn_b,
           high_dw_w, high_dw_bn_s, high_dw_bn_b,
           gather_w, gather_bn_s, gather_bn_b,
           low, high):
    N, Cl, Hl, Wl = low.shape
    _, Ch, Hh, Wh = high.shape
    mid_l = prog_low_w.shape[1]
    dim_out = gather_w.shape[1]

    wpw_low = (prog_low_w * prog_low_bn_s[None, :]).astype(jnp.bfloat16)
    wpw_high = (prog_high_w * prog_high_bn_s[None, :]).astype(jnp.bfloat16)
    wdw_low = low_dw_w * low_dw_bn_s[None, :]
    wdw_high = high_dw_w * high_dw_bn_s[None, :]
    wg = gather_w * gather_bn_s[None, :]
    wproj_low = wg[:mid_l].astype(jnp.bfloat16)
    wproj_high = wg[mid_l:].astype(jnp.bfloat16)
    bias_g = gather_bn_b.reshape(dim_out, 1)

    low3 = low.reshape(N, Cl, Hl * Wl)
    high3 = high.reshape(N, Ch, Hh * Wh)

    gl, gh = _run_branches(
        low3, high3,
        wpw_low, prog_low_bn_b.reshape(1, -1), wdw_low,
        low_dw_bn_b.reshape(1, -1), wproj_low, bias_g,
        wpw_high, prog_high_bn_b.reshape(1, -1), wdw_high,
        high_dw_bn_b.reshape(1, -1), wproj_high,
        Hl, Wl, Hh, Wh, dim_out)

    M = np.kron(_bilinear_matrix(Hl, Hh), _bilinear_matrix(Wl, Wh))
    MT = jnp.asarray(np.ascontiguousarray(M.T), dtype=jnp.bfloat16)

    out3 = _run_combine(gl, gh, MT, batch=4)
    return out3.reshape(N, dim_out, Hl, Wl)


# speedup vs baseline: 1.4862x; 1.0337x over previous
import functools

import numpy as np
import jax
import jax.numpy as jnp
from jax.experimental import pallas as pl
from jax.experimental.pallas import tpu as pltpu


def _bilinear_matrix(out_size, in_size):
    if out_size == 1 or in_size == 1:
        R = np.zeros((out_size, in_size), np.float32)
        R[:, 0] = 1.0
        return R
    src = np.arange(out_size, dtype=np.float64) * (in_size - 1) / (out_size - 1)
    lo = np.clip(np.floor(src).astype(np.int64), 0, in_size - 1)
    hi = np.clip(lo + 1, 0, in_size - 1)
    frac = (src - lo).astype(np.float32)
    R = np.zeros((out_size, in_size), np.float32)
    R[np.arange(out_size), lo] += 1.0 - frac
    R[np.arange(out_size), hi] += frac
    return R


def _branch(x_ref, wpw_ref, spw_ref, bpw_ref, wdw_ref, sdw_ref, bdw_ref,
            wg_ref, sg_ref, bg_ref, o_ref, pad_ref, sl_ref, sr_ref,
            *, H, W, g_lo, g_hi):
    C = wpw_ref.shape[1]
    wpw = (wpw_ref[...] * spw_ref[...]).astype(jnp.bfloat16)
    y = jax.lax.dot_general(
        x_ref[0].astype(jnp.bfloat16), wpw,
        dimension_numbers=(((0,), (0,)), ((), ())),
        preferred_element_type=jnp.float32)
    y = jnp.maximum(y + bpw_ref[...], 0.0)

    pad_ref[...] = jnp.zeros_like(pad_ref)
    pad_ref[pl.ds(1, H), pl.ds(8, W), :] = y.reshape(H, W, C)
    xp = pad_ref[...]

    sl_ref[...] = xp[:, 7:7 + W, :]
    sr_ref[...] = xp[:, 9:9 + W, :]
    left = sl_ref[...]
    right = sr_ref[...]
    wdw = wdw_ref[...] * sdw_ref[...]
    acc = jnp.zeros((H, W, C), jnp.float32)
    for dy in range(3):
        acc = acc + (left[dy:dy + H] * wdw[dy * 3 + 0]
                     + xp[dy:dy + H, 8:8 + W, :] * wdw[dy * 3 + 1]
                     + right[dy:dy + H] * wdw[dy * 3 + 2])
    z = jnp.maximum(acc + bdw_ref[0], 0.0)

    wproj = (wg_ref[g_lo:g_hi, :] * sg_ref[...]).astype(jnp.bfloat16)
    g = jax.lax.dot_general(
        wproj, z.reshape(H * W, C).astype(jnp.bfloat16),
        dimension_numbers=(((0,), (1,)), ((), ())),
        preferred_element_type=jnp.float32)
    if bg_ref is not None:
        g = g + jnp.transpose(bg_ref[...])
    o_ref[0] = g.astype(o_ref.dtype)


def _branches_kernel(low_ref, high_ref,
                     wpwl_ref, spwl_ref, bpwl_ref,
                     wdwl_ref, sdwl_ref, bdwl_ref,
                     wpwh_ref, spwh_ref, bpwh_ref,
                     wdwh_ref, sdwh_ref, bdwh_ref,
                     wg_ref, sg_ref, bg_ref,
                     gl_ref, gh_ref, padl_ref, padh_ref, sll_ref, srl_ref,
                     slh_ref, srh_ref, *, Hl, Wl, Hh, Wh, Cml):
    _branch(low_ref, wpwl_ref, spwl_ref, bpwl_ref, wdwl_ref, sdwl_ref,
            bdwl_ref, wg_ref, sg_ref, bg_ref, gl_ref, padl_ref, sll_ref,
            srl_ref, H=Hl, W=Wl, g_lo=0, g_hi=Cml)
    _branch(high_ref, wpwh_ref, spwh_ref, bpwh_ref, wdwh_ref, sdwh_ref,
            bdwh_ref, wg_ref, sg_ref, None, gh_ref, padh_ref, slh_ref,
            srh_ref, H=Hh, W=Wh, g_lo=Cml, g_hi=wg_ref.shape[0])


def _run_branches(low3, high3,
                  wl, sl, bl, dwl, sdl, dbl,
                  wh, sh, bh, dwh, sdh, dbh,
                  wg, sg, bg, Hl, Wl, Hh, Wh, Cout):
    N, Cl, HWl = low3.shape
    _, Ch, HWh = high3.shape
    Cml = wl.shape[1]
    Cmh = wh.shape[1]
    flops = int(N * (HWl * (2 * Cl * Cml + 18 * Cml + 2 * Cml * Cout)
                     + HWh * (2 * Ch * Cmh + 18 * Cmh + 2 * Cmh * Cout)))
    bytes_accessed = int(4 * (low3.size + high3.size)
                         + 4 * N * Cout * HWl + 2 * N * Cout * HWh)

    def cspec(*shape):
        return pl.BlockSpec(shape, lambda b: (0,) * len(shape))

    return pl.pallas_call(
        functools.partial(_branches_kernel, Hl=Hl, Wl=Wl, Hh=Hh, Wh=Wh,
                          Cml=Cml),
        out_shape=(jax.ShapeDtypeStruct((N, Cout, HWl), jnp.float32),
                   jax.ShapeDtypeStruct((N, Cout, HWh), jnp.bfloat16)),
        grid=(N,),
        in_specs=[
            pl.BlockSpec((1, Cl, HWl), lambda b: (b, 0, 0)),
            pl.BlockSpec((1, Ch, HWh), lambda b: (b, 0, 0)),
            cspec(Cl, Cml), cspec(1, Cml), cspec(1, Cml),
            cspec(9, Cml), cspec(1, Cml), cspec(1, Cml),
            cspec(Ch, Cmh), cspec(1, Cmh), cspec(1, Cmh),
            cspec(9, Cmh), cspec(1, Cmh), cspec(1, Cmh),
            cspec(Cml + Cmh, Cout), cspec(1, Cout), cspec(1, Cout),
        ],
        out_specs=(pl.BlockSpec((1, Cout, HWl), lambda b: (b, 0, 0)),
                   pl.BlockSpec((1, Cout, HWh), lambda b: (b, 0, 0))),
        scratch_shapes=[pltpu.VMEM((Hl + 2, Wl + 16, Cml), jnp.float32),
                        pltpu.VMEM((Hh + 2, Wh + 16, Cmh), jnp.float32),
                        pltpu.VMEM((Hl + 2, Wl, Cml), jnp.float32),
                        pltpu.VMEM((Hl + 2, Wl, Cml), jnp.float32),
                        pltpu.VMEM((Hh + 2, Wh, Cmh), jnp.float32),
                        pltpu.VMEM((Hh + 2, Wh, Cmh), jnp.float32)],
        compiler_params=pltpu.CompilerParams(
            dimension_semantics=("parallel",),
            vmem_limit_bytes=64 * 1024 * 1024),
        cost_estimate=pl.CostEstimate(flops=flops, transcendentals=0,
                                      bytes_accessed=bytes_accessed),
    )(low3, high3, wl, sl, bl, dwl, sdl, dbl,
      wh, sh, bh, dwh, sdh, dbh, wg, sg, bg)


def _combine_kernel(gl_ref, gh_ref, mt_ref, o_ref, *, B, Cout):
    HWh = gh_ref.shape[2]
    HWl = gl_ref.shape[2]
    gh = gh_ref[...].reshape(B * Cout, HWh)
    r = jnp.dot(gh, mt_ref[...], preferred_element_type=jnp.float32)
    o_ref[...] = (gl_ref[...].reshape(B * Cout, HWl) + r).reshape(
        B, Cout, HWl).astype(o_ref.dtype)


def _run_combine(gl, gh, MT, batch):
    N, Cout, HWl = gl.shape
    _, _, HWh = gh.shape
    flops = int(2 * N * Cout * HWh * HWl)
    bytes_accessed = int(4 * gl.size + 2 * gh.size + 2 * MT.size + 4 * gl.size)
    return pl.pallas_call(
        functools.partial(_combine_kernel, B=batch, Cout=Cout),
        out_shape=jax.ShapeDtypeStruct((N, Cout, HWl), jnp.float32),
        grid=(N // batch,),
        in_specs=[
            pl.BlockSpec((batch, Cout, HWl), lambda b: (b, 0, 0)),
            pl.BlockSpec((batch, Cout, HWh), lambda b: (b, 0, 0)),
            pl.BlockSpec((HWh, HWl), lambda b: (0, 0)),
        ],
        out_specs=pl.BlockSpec((batch, Cout, HWl), lambda b: (b, 0, 0)),
        compiler_params=pltpu.CompilerParams(
            dimension_semantics=("parallel",),
            vmem_limit_bytes=64 * 1024 * 1024),
        cost_estimate=pl.CostEstimate(flops=flops, transcendentals=0,
                                      bytes_accessed=bytes_accessed),
    )(gl, gh, MT)


def kernel(prog_low_w, prog_low_bn_s, prog_low_bn_b,
           low_dw_w, low_dw_bn_s, low_dw_bn_b,
           prog_high_w, prog_high_bn_s, prog_high_bn_b,
           high_dw_w, high_dw_bn_s, high_dw_bn_b,
           gather_w, gather_bn_s, gather_bn_b,
           low, high):
    N, Cl, Hl, Wl = low.shape
    _, Ch, Hh, Wh = high.shape
    dim_out = gather_w.shape[1]

    low3 = low.reshape(N, Cl, Hl * Wl)
    high3 = high.reshape(N, Ch, Hh * Wh)

    gl, gh = _run_branches(
        low3, high3,
        prog_low_w, prog_low_bn_s.reshape(1, -1), prog_low_bn_b.reshape(1, -1),
        low_dw_w, low_dw_bn_s.reshape(1, -1), low_dw_bn_b.reshape(1, -1),
        prog_high_w, prog_high_bn_s.reshape(1, -1),
        prog_high_bn_b.reshape(1, -1),
        high_dw_w, high_dw_bn_s.reshape(1, -1), high_dw_bn_b.reshape(1, -1),
        gather_w, gather_bn_s.reshape(1, -1), gather_bn_b.reshape(1, -1),
        Hl, Wl, Hh, Wh, dim_out)

    M = np.kron(_bilinear_matrix(Hl, Hh), _bilinear_matrix(Wl, Wh))
    MT = jnp.asarray(np.ascontiguousarray(M.T), dtype=jnp.bfloat16)

    out3 = _run_combine(gl, gh, MT, batch=4)
    return out3.reshape(N, dim_out, Hl, Wl)


# speedup vs baseline: 1.5993x; 1.0761x over previous
import functools

import numpy as np
import jax
import jax.numpy as jnp
from jax.experimental import pallas as pl
from jax.experimental.pallas import tpu as pltpu


def _bilinear_matrix(out_size, in_size):
    if out_size == 1 or in_size == 1:
        R = np.zeros((out_size, in_size), np.float32)
        R[:, 0] = 1.0
        return R
    src = np.arange(out_size, dtype=np.float64) * (in_size - 1) / (out_size - 1)
    lo = np.clip(np.floor(src).astype(np.int64), 0, in_size - 1)
    hi = np.clip(lo + 1, 0, in_size - 1)
    frac = (src - lo).astype(np.float32)
    R = np.zeros((out_size, in_size), np.float32)
    R[np.arange(out_size), lo] += 1.0 - frac
    R[np.arange(out_size), hi] += frac
    return R


def _branch(x_ref, wpw_ref, spw_ref, bpw_ref, wdw_ref, sdw_ref, bdw_ref,
            wg_ref, sg_ref, bg_ref, o_ref, pad_ref, sl_ref, sr_ref,
            *, H, W, g_lo, g_hi):
    C = wpw_ref.shape[1]
    wpw = (wpw_ref[...] * spw_ref[...]).astype(jnp.bfloat16)
    y = jax.lax.dot_general(
        x_ref[0].astype(jnp.bfloat16), wpw,
        dimension_numbers=(((0,), (0,)), ((), ())),
        preferred_element_type=jnp.float32)
    y = jnp.maximum(y + bpw_ref[...], 0.0)

    pad_ref[...] = jnp.zeros_like(pad_ref)
    pad_ref[pl.ds(1, H), pl.ds(8, W), :] = y.astype(jnp.bfloat16)
    xp = pad_ref[...]

    sl_ref[...] = xp[:, 7:7 + W, :]
    sr_ref[...] = xp[:, 9:9 + W, :]
    left = sl_ref[...]
    right = sr_ref[...]
    wdw = (wdw_ref[...] * sdw_ref[...]).astype(jnp.bfloat16)
    rows = []
    for dy in range(3):
        rows.append((left[dy:dy + H] * wdw[dy * 3 + 0]
                     + xp[dy:dy + H, 8:8 + W, :] * wdw[dy * 3 + 1])
                    + right[dy:dy + H] * wdw[dy * 3 + 2])
    acc = (rows[0] + rows[1]) + rows[2]
    z = jnp.maximum(acc + bdw_ref[...].astype(jnp.bfloat16), 0)

    wproj = (wg_ref[g_lo:g_hi, :] * sg_ref[...]).astype(jnp.bfloat16)
    g = jax.lax.dot_general(
        wproj, z.reshape(H * W, C),
        dimension_numbers=(((0,), (1,)), ((), ())),
        preferred_element_type=jnp.float32)
    if bg_ref is not None:
        g = g + jnp.transpose(bg_ref[...])
    o_ref[0] = g.astype(o_ref.dtype)


def _branches_kernel(low_ref, high_ref,
                     wpwl_ref, spwl_ref, bpwl_ref,
                     wdwl_ref, sdwl_ref, bdwl_ref,
                     wpwh_ref, spwh_ref, bpwh_ref,
                     wdwh_ref, sdwh_ref, bdwh_ref,
                     wg_ref, sg_ref, bg_ref,
                     gl_ref, gh_ref, padl_ref, padh_ref, sll_ref, srl_ref,
                     slh_ref, srh_ref, *, Hl, Wl, Hh, Wh, Cml):
    _branch(low_ref, wpwl_ref, spwl_ref, bpwl_ref, wdwl_ref, sdwl_ref,
            bdwl_ref, wg_ref, sg_ref, bg_ref, gl_ref, padl_ref, sll_ref,
            srl_ref, H=Hl, W=Wl, g_lo=0, g_hi=Cml)
    _branch(high_ref, wpwh_ref, spwh_ref, bpwh_ref, wdwh_ref, sdwh_ref,
            bdwh_ref, wg_ref, sg_ref, None, gh_ref, padh_ref, slh_ref,
            srh_ref, H=Hh, W=Wh, g_lo=Cml, g_hi=wg_ref.shape[0])


def _run_branches(low4, high4,
                  wl, sl, bl, dwl, sdl, dbl,
                  wh, sh, bh, dwh, sdh, dbh,
                  wg, sg, bg, Cout):
    N, Cl, Hl, Wl = low4.shape
    _, Ch, Hh, Wh = high4.shape
    HWl = Hl * Wl
    HWh = Hh * Wh
    Cml = wl.shape[1]
    Cmh = wh.shape[1]
    flops = int(N * (HWl * (2 * Cl * Cml + 18 * Cml + 2 * Cml * Cout)
                     + HWh * (2 * Ch * Cmh + 18 * Cmh + 2 * Cmh * Cout)))
    bytes_accessed = int(4 * (low4.size + high4.size)
                         + 4 * N * Cout * HWl + 2 * N * Cout * HWh)

    def cspec(*shape):
        return pl.BlockSpec(shape, lambda b: (0,) * len(shape))

    return pl.pallas_call(
        functools.partial(_branches_kernel, Hl=Hl, Wl=Wl, Hh=Hh, Wh=Wh,
                          Cml=Cml),
        out_shape=(jax.ShapeDtypeStruct((N, Cout, HWl), jnp.float32),
                   jax.ShapeDtypeStruct((N, Cout, HWh), jnp.bfloat16)),
        grid=(N,),
        in_specs=[
            pl.BlockSpec((1, Cl, Hl, Wl), lambda b: (b, 0, 0, 0)),
            pl.BlockSpec((1, Ch, Hh, Wh), lambda b: (b, 0, 0, 0)),
            cspec(Cl, Cml), cspec(1, Cml), cspec(1, Cml),
            cspec(9, Cml), cspec(1, Cml), cspec(1, Cml),
            cspec(Ch, Cmh), cspec(1, Cmh), cspec(1, Cmh),
            cspec(9, Cmh), cspec(1, Cmh), cspec(1, Cmh),
            cspec(Cml + Cmh, Cout), cspec(1, Cout), cspec(1, Cout),
        ],
        out_specs=(pl.BlockSpec((1, Cout, HWl), lambda b: (b, 0, 0)),
                   pl.BlockSpec((1, Cout, HWh), lambda b: (b, 0, 0))),
        scratch_shapes=[pltpu.VMEM((Hl + 2, Wl + 16, Cml), jnp.bfloat16),
                        pltpu.VMEM((Hh + 2, Wh + 16, Cmh), jnp.bfloat16),
                        pltpu.VMEM((Hl + 2, Wl, Cml), jnp.bfloat16),
                        pltpu.VMEM((Hl + 2, Wl, Cml), jnp.bfloat16),
                        pltpu.VMEM((Hh + 2, Wh, Cmh), jnp.bfloat16),
                        pltpu.VMEM((Hh + 2, Wh, Cmh), jnp.bfloat16)],
        compiler_params=pltpu.CompilerParams(
            dimension_semantics=("arbitrary",),
            vmem_limit_bytes=64 * 1024 * 1024),
        cost_estimate=pl.CostEstimate(flops=flops, transcendentals=0,
                                      bytes_accessed=bytes_accessed),
    )(low4, high4, wl, sl, bl, dwl, sdl, dbl,
      wh, sh, bh, dwh, sdh, dbh, wg, sg, bg)


def _combine_kernel(gl_ref, gh_ref, mt_ref, o_ref, *, B, Cout):
    HWh = gh_ref.shape[2]
    HWl = gl_ref.shape[2]
    gh = gh_ref[...].reshape(B * Cout, HWh)
    r = jnp.dot(gh, mt_ref[...], preferred_element_type=jnp.float32)
    o_ref[...] = (gl_ref[...].reshape(B * Cout, HWl) + r).reshape(
        B, Cout, HWl).astype(o_ref.dtype)


def _run_combine(gl, gh, MT, batch):
    N, Cout, HWl = gl.shape
    _, _, HWh = gh.shape
    flops = int(2 * N * Cout * HWh * HWl)
    bytes_accessed = int(4 * gl.size + 2 * gh.size + 2 * MT.size + 4 * gl.size)
    return pl.pallas_call(
        functools.partial(_combine_kernel, B=batch, Cout=Cout),
        out_shape=jax.ShapeDtypeStruct((N, Cout, HWl), jnp.float32),
        grid=(N // batch,),
        in_specs=[
            pl.BlockSpec((batch, Cout, HWl), lambda b: (b, 0, 0)),
            pl.BlockSpec((batch, Cout, HWh), lambda b: (b, 0, 0)),
            pl.BlockSpec((HWh, HWl), lambda b: (0, 0)),
        ],
        out_specs=pl.BlockSpec((batch, Cout, HWl), lambda b: (b, 0, 0)),
        compiler_params=pltpu.CompilerParams(
            dimension_semantics=("arbitrary",),
            vmem_limit_bytes=64 * 1024 * 1024),
        cost_estimate=pl.CostEstimate(flops=flops, transcendentals=0,
                                      bytes_accessed=bytes_accessed),
    )(gl, gh, MT)


def kernel(prog_low_w, prog_low_bn_s, prog_low_bn_b,
           low_dw_w, low_dw_bn_s, low_dw_bn_b,
           prog_high_w, prog_high_bn_s, prog_high_bn_b,
           high_dw_w, high_dw_bn_s, high_dw_bn_b,
           gather_w, gather_bn_s, gather_bn_b,
           low, high):
    N, Cl, Hl, Wl = low.shape
    _, Ch, Hh, Wh = high.shape
    dim_out = gather_w.shape[1]

    gl, gh = _run_branches(
        low, high,
        prog_low_w, prog_low_bn_s.reshape(1, -1), prog_low_bn_b.reshape(1, -1),
        low_dw_w, low_dw_bn_s.reshape(1, -1), low_dw_bn_b.reshape(1, -1),
        prog_high_w, prog_high_bn_s.reshape(1, -1),
        prog_high_bn_b.reshape(1, -1),
        high_dw_w, high_dw_bn_s.reshape(1, -1), high_dw_bn_b.reshape(1, -1),
        gather_w, gather_bn_s.reshape(1, -1), gather_bn_b.reshape(1, -1),
        dim_out)

    M = np.kron(_bilinear_matrix(Hl, Hh), _bilinear_matrix(Wl, Wh))
    MT = jnp.asarray(np.ascontiguousarray(M.T), dtype=jnp.bfloat16)

    out3 = _run_combine(gl, gh, MT, batch=min(4, N // 2))
    return out3.reshape(N, dim_out, Hl, Wl)


# speedup vs baseline: 1.6038x; 1.0028x over previous
import functools

import numpy as np
import jax
import jax.numpy as jnp
from jax.experimental import pallas as pl
from jax.experimental.pallas import tpu as pltpu


def _bilinear_matrix(out_size, in_size):
    if out_size == 1 or in_size == 1:
        R = np.zeros((out_size, in_size), np.float32)
        R[:, 0] = 1.0
        return R
    src = np.arange(out_size, dtype=np.float64) * (in_size - 1) / (out_size - 1)
    lo = np.clip(np.floor(src).astype(np.int64), 0, in_size - 1)
    hi = np.clip(lo + 1, 0, in_size - 1)
    frac = (src - lo).astype(np.float32)
    R = np.zeros((out_size, in_size), np.float32)
    R[np.arange(out_size), lo] += 1.0 - frac
    R[np.arange(out_size), hi] += frac
    return R


def _branch(x_ref, wpw_ref, spw_ref, bpw_ref, wdw_ref, sdw_ref, bdw_ref,
            wg_ref, sg_ref, bg_ref, o_ref, pad_ref, sl_ref, sr_ref,
            *, H, W, g_lo, g_hi):
    C = wpw_ref.shape[1]
    wpw = (wpw_ref[...] * spw_ref[...]).astype(jnp.bfloat16)
    y = jax.lax.dot_general(
        x_ref[0].astype(jnp.bfloat16), wpw,
        dimension_numbers=(((0,), (0,)), ((), ())),
        preferred_element_type=jnp.float32)
    y = jnp.maximum(y + bpw_ref[...], 0.0)

    pad_ref[...] = jnp.zeros_like(pad_ref)
    pad_ref[pl.ds(1, H), pl.ds(8, W), :] = y.astype(jnp.bfloat16)
    xp = pad_ref[...]

    sl_ref[...] = xp[:, 7:7 + W, :]
    sr_ref[...] = xp[:, 9:9 + W, :]
    left = sl_ref[...]
    right = sr_ref[...]
    wdw = (wdw_ref[...] * sdw_ref[...]).astype(jnp.bfloat16)
    rows = []
    for dy in range(3):
        rows.append((left[dy:dy + H] * wdw[dy * 3 + 0]
                     + xp[dy:dy + H, 8:8 + W, :] * wdw[dy * 3 + 1])
                    + right[dy:dy + H] * wdw[dy * 3 + 2])
    acc = (rows[0] + rows[1]) + rows[2]
    z = jnp.maximum(acc + bdw_ref[...].astype(jnp.bfloat16), 0)

    wproj = (wg_ref[g_lo:g_hi, :] * sg_ref[...]).astype(jnp.bfloat16)
    g = jax.lax.dot_general(
        wproj, z.reshape(H * W, C),
        dimension_numbers=(((0,), (1,)), ((), ())),
        preferred_element_type=jnp.float32)
    if bg_ref is not None:
        g = g + jnp.transpose(bg_ref[...])
    o_ref[0] = g.astype(o_ref.dtype)


def _branches_kernel(low_ref, high_ref,
                     wpwl_ref, spwl_ref, bpwl_ref,
                     wdwl_ref, sdwl_ref, bdwl_ref,
                     wpwh_ref, spwh_ref, bpwh_ref,
                     wdwh_ref, sdwh_ref, bdwh_ref,
                     wg_ref, sg_ref, bg_ref,
                     gl_ref, gh_ref, padl_ref, padh_ref, sll_ref, srl_ref,
                     slh_ref, srh_ref, *, Hl, Wl, Hh, Wh, Cml):
    _branch(low_ref, wpwl_ref, spwl_ref, bpwl_ref, wdwl_ref, sdwl_ref,
            bdwl_ref, wg_ref, sg_ref, bg_ref, gl_ref, padl_ref, sll_ref,
            srl_ref, H=Hl, W=Wl, g_lo=0, g_hi=Cml)
    _branch(high_ref, wpwh_ref, spwh_ref, bpwh_ref, wdwh_ref, sdwh_ref,
            bdwh_ref, wg_ref, sg_ref, None, gh_ref, padh_ref, slh_ref,
            srh_ref, H=Hh, W=Wh, g_lo=Cml, g_hi=wg_ref.shape[0])


def _run_branches(low4, high4,
                  wl, sl, bl, dwl, sdl, dbl,
                  wh, sh, bh, dwh, sdh, dbh,
                  wg, sg, bg, Cout):
    N, Cl, Hl, Wl = low4.shape
    _, Ch, Hh, Wh = high4.shape
    HWl = Hl * Wl
    HWh = Hh * Wh
    Cml = wl.shape[1]
    Cmh = wh.shape[1]
    flops = int(N * (HWl * (2 * Cl * Cml + 18 * Cml + 2 * Cml * Cout)
                     + HWh * (2 * Ch * Cmh + 18 * Cmh + 2 * Cmh * Cout)))
    bytes_accessed = int(4 * (low4.size + high4.size)
                         + 4 * N * Cout * HWl + 2 * N * Cout * HWh)

    def cspec(*shape):
        return pl.BlockSpec(shape, lambda b: (0,) * len(shape))

    return pl.pallas_call(
        functools.partial(_branches_kernel, Hl=Hl, Wl=Wl, Hh=Hh, Wh=Wh,
                          Cml=Cml),
        out_shape=(jax.ShapeDtypeStruct((N, Cout, HWl), jnp.bfloat16),
                   jax.ShapeDtypeStruct((N, Cout, HWh), jnp.bfloat16)),
        grid=(N,),
        in_specs=[
            pl.BlockSpec((1, Cl, Hl, Wl), lambda b: (b, 0, 0, 0)),
            pl.BlockSpec((1, Ch, Hh, Wh), lambda b: (b, 0, 0, 0)),
            cspec(Cl, Cml), cspec(1, Cml), cspec(1, Cml),
            cspec(9, Cml), cspec(1, Cml), cspec(1, Cml),
            cspec(Ch, Cmh), cspec(1, Cmh), cspec(1, Cmh),
            cspec(9, Cmh), cspec(1, Cmh), cspec(1, Cmh),
            cspec(Cml + Cmh, Cout), cspec(1, Cout), cspec(1, Cout),
        ],
        out_specs=(pl.BlockSpec((1, Cout, HWl), lambda b: (b, 0, 0)),
                   pl.BlockSpec((1, Cout, HWh), lambda b: (b, 0, 0))),
        scratch_shapes=[pltpu.VMEM((Hl + 2, Wl + 16, Cml), jnp.bfloat16),
                        pltpu.VMEM((Hh + 2, Wh + 16, Cmh), jnp.bfloat16),
                        pltpu.VMEM((Hl + 2, Wl, Cml), jnp.bfloat16),
                        pltpu.VMEM((Hl + 2, Wl, Cml), jnp.bfloat16),
                        pltpu.VMEM((Hh + 2, Wh, Cmh), jnp.bfloat16),
                        pltpu.VMEM((Hh + 2, Wh, Cmh), jnp.bfloat16)],
        compiler_params=pltpu.CompilerParams(
            dimension_semantics=("arbitrary",),
            vmem_limit_bytes=64 * 1024 * 1024),
        cost_estimate=pl.CostEstimate(flops=flops, transcendentals=0,
                                      bytes_accessed=bytes_accessed),
    )(low4, high4, wl, sl, bl, dwl, sdl, dbl,
      wh, sh, bh, dwh, sdh, dbh, wg, sg, bg)


def _combine_kernel(gl_ref, gh_ref, mt_ref, o_ref, *, B, Cout):
    HWh = gh_ref.shape[2]
    HWl = gl_ref.shape[2]
    gh = gh_ref[...].reshape(B * Cout, HWh)
    r = jnp.dot(gh, mt_ref[...], preferred_element_type=jnp.float32)
    o_ref[...] = (r + gl_ref[...].reshape(B * Cout, HWl).astype(jnp.float32)
                  ).reshape(B, Cout, HWl).astype(o_ref.dtype)


def _run_combine(gl, gh, MT, batch):
    N, Cout, HWl = gl.shape
    _, _, HWh = gh.shape
    flops = int(2 * N * Cout * HWh * HWl)
    bytes_accessed = int(4 * gl.size + 2 * gh.size + 2 * MT.size + 4 * gl.size)
    return pl.pallas_call(
        functools.partial(_combine_kernel, B=batch, Cout=Cout),
        out_shape=jax.ShapeDtypeStruct((N, Cout, HWl), jnp.float32),
        grid=(N // batch,),
        in_specs=[
            pl.BlockSpec((batch, Cout, HWl), lambda b: (b, 0, 0)),
            pl.BlockSpec((batch, Cout, HWh), lambda b: (b, 0, 0)),
            pl.BlockSpec((HWh, HWl), lambda b: (0, 0)),
        ],
        out_specs=pl.BlockSpec((batch, Cout, HWl), lambda b: (b, 0, 0)),
        compiler_params=pltpu.CompilerParams(
            dimension_semantics=("arbitrary",),
            vmem_limit_bytes=64 * 1024 * 1024),
        cost_estimate=pl.CostEstimate(flops=flops, transcendentals=0,
                                      bytes_accessed=bytes_accessed),
    )(gl, gh, MT)


def kernel(prog_low_w, prog_low_bn_s, prog_low_bn_b,
           low_dw_w, low_dw_bn_s, low_dw_bn_b,
           prog_high_w, prog_high_bn_s, prog_high_bn_b,
           high_dw_w, high_dw_bn_s, high_dw_bn_b,
           gather_w, gather_bn_s, gather_bn_b,
           low, high):
    N, Cl, Hl, Wl = low.shape
    _, Ch, Hh, Wh = high.shape
    dim_out = gather_w.shape[1]

    gl, gh = _run_branches(
        low, high,
        prog_low_w, prog_low_bn_s.reshape(1, -1), prog_low_bn_b.reshape(1, -1),
        low_dw_w, low_dw_bn_s.reshape(1, -1), low_dw_bn_b.reshape(1, -1),
        prog_high_w, prog_high_bn_s.reshape(1, -1),
        prog_high_bn_b.reshape(1, -1),
        high_dw_w, high_dw_bn_s.reshape(1, -1), high_dw_bn_b.reshape(1, -1),
        gather_w, gather_bn_s.reshape(1, -1), gather_bn_b.reshape(1, -1),
        dim_out)

    M = np.kron(_bilinear_matrix(Hl, Hh), _bilinear_matrix(Wl, Wh))
    MT = jnp.asarray(np.ascontiguousarray(M.T), dtype=jnp.bfloat16)

    out3 = _run_combine(gl, gh, MT, batch=min(4, N // 2))
    return out3.reshape(N, dim_out, Hl, Wl)
